# baseline (speedup 1.0000x reference)
import functools

import jax
import jax.numpy as jnp
from jax.experimental import pallas as pl
from jax.experimental.pallas import tpu as pltpu

T = 2048
D = 1024
F = 4096
E = 8
K = 2

BLK = 256
P = K * T + E * BLK
NB = P // BLK
FB = 512
NJ = F // FB

_SQRT_HALF = 0.7071067811865476


def _erf(z):
    a1, a2, a3, a4, a5 = (0.254829592, -0.284496736, 1.421413741,
                          -1.453152027, 1.061405429)
    s = jnp.sign(z)
    za = jnp.abs(z)
    t = 1.0 / (1.0 + 0.3275911 * za)
    poly = t * (a1 + t * (a2 + t * (a3 + t * (a4 + t * a5))))
    return s * (1.0 - poly * jnp.exp(-za * za))


def _gelu(h):
    return 0.5 * h * (1.0 + _erf(h * _SQRT_HALF))


def _ffn_body(be_ref, xs_ref, w1_ref, b1_ref, w2_ref, b2_ref, out_ref):
    j = pl.program_id(1)

    @pl.when(j == 0)
    def _init():
        out_ref[...] = jnp.broadcast_to(b2_ref[0, 0], out_ref.shape)

    h = jnp.dot(xs_ref[...], w1_ref[0], preferred_element_type=jnp.float32)
    h = _gelu(h + b1_ref[0, 0]).astype(jnp.bfloat16)
    out_ref[...] += jnp.dot(h, w2_ref[0], preferred_element_type=jnp.float32)


@functools.partial(jax.jit, static_argnames=())
def _ffn(xs, block_e, W1, b1, W2, b2):
    grid_spec = pltpu.PrefetchScalarGridSpec(
        num_scalar_prefetch=1,
        grid=(NB, NJ),
        in_specs=[
            pl.BlockSpec((BLK, D), lambda i, j, be: (i, 0)),
            pl.BlockSpec((1, D, FB), lambda i, j, be: (be[i], 0, j)),
            pl.BlockSpec((1, 1, FB), lambda i, j, be: (be[i], 0, j)),
            pl.BlockSpec((1, FB, D), lambda i, j, be: (be[i], j, 0)),
            pl.BlockSpec((1, 1, D), lambda i, j, be: (be[i], 0, 0)),
        ],
        out_specs=pl.BlockSpec((BLK, D), lambda i, j, be: (i, 0)),
    )
    return pl.pallas_call(
        _ffn_body,
        grid_spec=grid_spec,
        out_shape=jax.ShapeDtypeStruct((P, D), jnp.float32),
        compiler_params=pltpu.CompilerParams(
            dimension_semantics=("arbitrary", "arbitrary"),
        ),
    )(block_e, xs, W1.astype(jnp.bfloat16), b1.reshape(E, 1, F),
      W2.astype(jnp.bfloat16), b2.reshape(E, 1, D))


def kernel(x, gate_W, W1, b1, W2, b2):
    logits = x @ gate_W
    probs = jax.nn.softmax(logits, axis=-1)
    top_p, top_i = jax.lax.top_k(probs, K)
    top_w = top_p / jnp.sum(top_p, axis=-1, keepdims=True)

    ef = top_i.reshape(-1).astype(jnp.int32)
    order = jnp.argsort(ef)
    counts = jnp.sum(ef[None, :] == jnp.arange(E, dtype=jnp.int32)[:, None],
                     axis=1).astype(jnp.int32)
    blocks_e = (counts + BLK - 1) // BLK
    first_block = jnp.concatenate(
        [jnp.zeros((1,), jnp.int32), jnp.cumsum(blocks_e)[:-1]])
    pad_start = first_block * BLK
    seg_start = jnp.concatenate(
        [jnp.zeros((1,), jnp.int32), jnp.cumsum(counts)[:-1]])

    r = jnp.arange(K * T, dtype=jnp.int32)
    e_sorted = ef[order]
    padded_row = pad_start[e_sorted] + (r - seg_start[e_sorted])

    pos = jnp.zeros((K * T,), jnp.int32).at[order].set(padded_row)
    rowtok = jnp.zeros((P,), jnp.int32).at[padded_row].set(order // K)

    block_e = (jnp.sum(jnp.arange(NB, dtype=jnp.int32)[:, None]
                       >= first_block[None, :], axis=1) - 1).astype(jnp.int32)

    xs = jnp.take(x.astype(jnp.bfloat16), rowtok, axis=0)
    ys = _ffn(xs, block_e, W1, b1, W2, b2)

    pos2 = pos.reshape(T, K)
    out = (jnp.take(ys, pos2[:, 0], axis=0) * top_w[:, 0:1]
           + jnp.take(ys, pos2[:, 1], axis=0) * top_w[:, 1:2])
    return out


# speedup vs baseline: 1.3563x; 1.3563x over previous
import functools

import jax
import jax.numpy as jnp
from jax.experimental import pallas as pl
from jax.experimental.pallas import tpu as pltpu

T = 2048
D = 1024
F = 4096
E = 8
K = 2

BLK = 256
P = K * T + E * BLK
NB = P // BLK
FB = 2048
NJ = F // FB

_SQRT_HALF = 0.7071067811865476


def _erf(z):
    a1, a2, a3, a4, a5 = (0.254829592, -0.284496736, 1.421413741,
                          -1.453152027, 1.061405429)
    s = jnp.sign(z)
    za = jnp.abs(z)
    t = 1.0 / (1.0 + 0.3275911 * za)
    poly = t * (a1 + t * (a2 + t * (a3 + t * (a4 + t * a5))))
    return s * (1.0 - poly * jnp.exp(-za * za))


def _gelu(h):
    return 0.5 * h * (1.0 + _erf(h * _SQRT_HALF))


def _ffn_body(be_ref, xs_ref, w1_ref, b1_ref, w2_ref, b2_ref, out_ref):
    j = pl.program_id(0)
    w1 = w1_ref[0].astype(jnp.bfloat16)
    w2 = w2_ref[0].astype(jnp.bfloat16)
    h = jnp.dot(xs_ref[...], w1, preferred_element_type=jnp.float32)
    h = _gelu(h + b1_ref[0, 0]).astype(jnp.bfloat16)
    y = jnp.dot(h, w2, preferred_element_type=jnp.float32)
    out_ref[0] = jnp.where(j == 0, y + b2_ref[0, 0], y)


@jax.jit
def _ffn(xs, block_e, W1, b1, W2, b2):
    grid_spec = pltpu.PrefetchScalarGridSpec(
        num_scalar_prefetch=1,
        grid=(NJ, NB),
        in_specs=[
            pl.BlockSpec((BLK, D), lambda j, i, be: (i, 0)),
            pl.BlockSpec((1, D, FB), lambda j, i, be: (be[i], 0, j)),
            pl.BlockSpec((1, 1, FB), lambda j, i, be: (be[i], 0, j)),
            pl.BlockSpec((1, FB, D), lambda j, i, be: (be[i], j, 0)),
            pl.BlockSpec((1, 1, D), lambda j, i, be: (be[i], 0, 0)),
        ],
        out_specs=pl.BlockSpec((1, BLK, D), lambda j, i, be: (j, i, 0)),
    )
    return pl.pallas_call(
        _ffn_body,
        grid_spec=grid_spec,
        out_shape=jax.ShapeDtypeStruct((NJ, P, D), jnp.float32),
        compiler_params=pltpu.CompilerParams(
            dimension_semantics=("arbitrary", "arbitrary"),
        ),
    )(block_e, xs, W1, b1.reshape(E, 1, F), W2, b2.reshape(E, 1, D))


def kernel(x, gate_W, W1, b1, W2, b2):
    logits = x @ gate_W
    probs = jax.nn.softmax(logits, axis=-1)
    top_p, top_i = jax.lax.top_k(probs, K)
    top_w = top_p / jnp.sum(top_p, axis=-1, keepdims=True)

    ef = top_i.reshape(-1).astype(jnp.int32)
    order = jnp.argsort(ef)
    counts = jnp.sum(ef[None, :] == jnp.arange(E, dtype=jnp.int32)[:, None],
                     axis=1).astype(jnp.int32)
    blocks_e = (counts + BLK - 1) // BLK
    first_block = jnp.concatenate(
        [jnp.zeros((1,), jnp.int32), jnp.cumsum(blocks_e)[:-1]])
    pad_start = first_block * BLK
    seg_start = jnp.concatenate(
        [jnp.zeros((1,), jnp.int32), jnp.cumsum(counts)[:-1]])

    r = jnp.arange(K * T, dtype=jnp.int32)
    e_sorted = ef[order]
    padded_row = pad_start[e_sorted] + (r - seg_start[e_sorted])

    pos = jnp.zeros((K * T,), jnp.int32).at[order].set(padded_row)
    rowtok = jnp.zeros((P,), jnp.int32).at[padded_row].set(order // K)

    block_e = (jnp.sum(jnp.arange(NB, dtype=jnp.int32)[:, None]
                       >= first_block[None, :], axis=1) - 1).astype(jnp.int32)

    xs = jnp.take(x.astype(jnp.bfloat16), rowtok, axis=0)
    yp = _ffn(xs, block_e, W1, b1, W2, b2)
    ys = jnp.sum(yp, axis=0)

    pos2 = pos.reshape(T, K)
    out = (jnp.take(ys, pos2[:, 0], axis=0) * top_w[:, 0:1]
           + jnp.take(ys, pos2[:, 1], axis=0) * top_w[:, 1:2])
    return out
